# Initial kernel scaffold; baseline (speedup 1.0000x reference)
import functools

import jax
import jax.numpy as jnp
from jax import lax
from jax.experimental import pallas as pl
from jax.experimental.pallas import tpu as pltpu
from jax.experimental.pallas import tpu_sc as plsc

N = 16384
M = 16
MAX_LENGTH = 20
TOKEN_DIM = 128
SPATIAL_DIM = 125
BLOCK_N = 512


def _sc_gather(node_neighbors, spatial_table):
    mesh = plsc.VectorSubcoreMesh(core_axis_name="c", subcore_axis_name="s")

    @functools.partial(
        pl.kernel,
        mesh=mesh,
        out_type=jax.ShapeDtypeStruct((M, SPATIAL_DIM), jnp.float32),
        scratch_types=[
            pltpu.VMEM((M,), jnp.int32),
            pltpu.VMEM((M, SPATIAL_DIM), jnp.float32),
            pltpu.SemaphoreType.DMA,
        ],
    )
    def gather_kernel(idx_hbm, table_hbm, out_hbm, idx_v, rows_v, sem):
        @pl.when((lax.axis_index("c") == 0) & (lax.axis_index("s") == 0))
        def _():
            pltpu.sync_copy(idx_hbm, idx_v)
            pltpu.async_copy(table_hbm.at[idx_v], rows_v, sem).wait()
            pltpu.sync_copy(rows_v, out_hbm)

    return gather_kernel(node_neighbors, spatial_table)


def _assemble_body(sp_ref, val_ref, tim_ref, w_ref, b_ref, out_ref):
    b = out_ref.shape[0]
    sp = sp_ref[...]
    sp128 = jnp.concatenate(
        [sp, jnp.zeros((M, TOKEN_DIM - SPATIAL_DIM), jnp.float32)], axis=1
    )
    tpl = jnp.concatenate(
        [sp128, jnp.zeros((MAX_LENGTH - M, TOKEN_DIM), jnp.float32)], axis=0
    )
    w = w_ref[0, 0]
    bias = b_ref[0, 0]
    val = val_ref[...] * w + bias
    val20 = jnp.concatenate(
        [val, jnp.zeros((b, MAX_LENGTH - M), jnp.float32)], axis=1
    )
    tim = tim_ref[...]

    lane = lax.broadcasted_iota(jnp.int32, (b, MAX_LENGTH, TOKEN_DIM), 2)
    row = lax.broadcasted_iota(jnp.int32, (b, MAX_LENGTH, TOKEN_DIM), 1)
    out = jnp.broadcast_to(tpl[None], (b, MAX_LENGTH, TOKEN_DIM))
    val_b = jnp.broadcast_to(val20[:, :, None], (b, MAX_LENGTH, TOKEN_DIM))
    t0_b = jnp.broadcast_to(tim[:, 0][:, None, None], (b, MAX_LENGTH, TOKEN_DIM))
    t1_b = jnp.broadcast_to(tim[:, 1][:, None, None], (b, MAX_LENGTH, TOKEN_DIM))
    valid = row < M
    out = jnp.where((lane == SPATIAL_DIM) & valid, val_b, out)
    out = jnp.where((lane == SPATIAL_DIM + 1) & valid, t0_b, out)
    out = jnp.where((lane == SPATIAL_DIM + 2) & valid, t1_b, out)
    out_ref[...] = out


def kernel(values, tim_emb, spatial_table, w_val, b_val, node_neighbors):
    sp = _sc_gather(node_neighbors, spatial_table)
    n = values.shape[0]
    grid = (n // BLOCK_N,)
    w2 = jnp.reshape(w_val, (1, 1))
    b2 = jnp.reshape(b_val, (1, 1))
    out = pl.pallas_call(
        _assemble_body,
        grid=grid,
        in_specs=[
            pl.BlockSpec((M, SPATIAL_DIM), lambda i: (0, 0)),
            pl.BlockSpec((BLOCK_N, M), lambda i: (i, 0)),
            pl.BlockSpec((BLOCK_N, 2), lambda i: (i, 0)),
            pl.BlockSpec(memory_space=pltpu.SMEM),
            pl.BlockSpec(memory_space=pltpu.SMEM),
        ],
        out_specs=pl.BlockSpec(
            (BLOCK_N, MAX_LENGTH, TOKEN_DIM), lambda i: (i, 0, 0)
        ),
        out_shape=jax.ShapeDtypeStruct((n, MAX_LENGTH, TOKEN_DIM), jnp.float32),
        compiler_params=pltpu.CompilerParams(
            dimension_semantics=("arbitrary",),
        ),
    )(sp, values, tim_emb, w2, b2)
    return out


# speedup vs baseline: 1.5498x; 1.5498x over previous
import functools

import jax
import jax.numpy as jnp
from jax import lax
from jax.experimental import pallas as pl
from jax.experimental.pallas import tpu as pltpu
from jax.experimental.pallas import tpu_sc as plsc

N = 16384
M = 16
MAX_LENGTH = 20
TOKEN_DIM = 128
SPATIAL_DIM = 125
BLOCK_N = 512


def _sc_gather(node_neighbors, spatial_table):
    mesh = plsc.VectorSubcoreMesh(core_axis_name="c", subcore_axis_name="s")

    @functools.partial(
        pl.kernel,
        mesh=mesh,
        out_type=jax.ShapeDtypeStruct((M, SPATIAL_DIM), jnp.float32),
        scratch_types=[
            pltpu.VMEM((M,), jnp.int32),
            pltpu.VMEM((M, SPATIAL_DIM), jnp.float32),
            pltpu.SemaphoreType.DMA,
        ],
    )
    def gather_kernel(idx_hbm, table_hbm, out_hbm, idx_v, rows_v, sem):
        @pl.when((lax.axis_index("c") == 0) & (lax.axis_index("s") == 0))
        def _():
            pltpu.sync_copy(idx_hbm, idx_v)
            iv = idx_v[...]
            copies = []
            for j in range(M):
                idx_j = iv[j]
                copies.append(
                    pltpu.async_copy(
                        table_hbm.at[pl.ds(idx_j, 1), :],
                        rows_v.at[pl.ds(j, 1), :],
                        sem,
                    )
                )
            for c in copies:
                c.wait()
            pltpu.sync_copy(rows_v, out_hbm)

    return gather_kernel(node_neighbors, spatial_table)


def _assemble_body(sp_ref, val_ref, tim_ref, w_ref, b_ref, out_ref):
    b = out_ref.shape[0]
    sp = sp_ref[...]
    sp128 = jnp.concatenate(
        [sp, jnp.zeros((M, TOKEN_DIM - SPATIAL_DIM), jnp.float32)], axis=1
    )
    tpl = jnp.concatenate(
        [sp128, jnp.zeros((MAX_LENGTH - M, TOKEN_DIM), jnp.float32)], axis=0
    )
    w = w_ref[0, 0]
    bias = b_ref[0, 0]
    val = val_ref[...] * w + bias
    val20 = jnp.concatenate(
        [val, jnp.zeros((b, MAX_LENGTH - M), jnp.float32)], axis=1
    )
    tim = tim_ref[...]

    lane = lax.broadcasted_iota(jnp.int32, (b, MAX_LENGTH, TOKEN_DIM), 2)
    row = lax.broadcasted_iota(jnp.int32, (b, MAX_LENGTH, TOKEN_DIM), 1)
    out = jnp.broadcast_to(tpl[None], (b, MAX_LENGTH, TOKEN_DIM))
    val_b = jnp.broadcast_to(val20[:, :, None], (b, MAX_LENGTH, TOKEN_DIM))
    t0_b = jnp.broadcast_to(tim[:, 0][:, None, None], (b, MAX_LENGTH, TOKEN_DIM))
    t1_b = jnp.broadcast_to(tim[:, 1][:, None, None], (b, MAX_LENGTH, TOKEN_DIM))
    valid = row < M
    out = jnp.where((lane == SPATIAL_DIM) & valid, val_b, out)
    out = jnp.where((lane == SPATIAL_DIM + 1) & valid, t0_b, out)
    out = jnp.where((lane == SPATIAL_DIM + 2) & valid, t1_b, out)
    out_ref[...] = out


def kernel(values, tim_emb, spatial_table, w_val, b_val, node_neighbors):
    sp = _sc_gather(node_neighbors, spatial_table)
    n = values.shape[0]
    grid = (n // BLOCK_N,)
    w2 = jnp.reshape(w_val, (1, 1))
    b2 = jnp.reshape(b_val, (1, 1))
    out = pl.pallas_call(
        _assemble_body,
        grid=grid,
        in_specs=[
            pl.BlockSpec((M, SPATIAL_DIM), lambda i: (0, 0)),
            pl.BlockSpec((BLOCK_N, M), lambda i: (i, 0)),
            pl.BlockSpec((BLOCK_N, 2), lambda i: (i, 0)),
            pl.BlockSpec(memory_space=pltpu.SMEM),
            pl.BlockSpec(memory_space=pltpu.SMEM),
        ],
        out_specs=pl.BlockSpec(
            (BLOCK_N, MAX_LENGTH, TOKEN_DIM), lambda i: (i, 0, 0)
        ),
        out_shape=jax.ShapeDtypeStruct((n, MAX_LENGTH, TOKEN_DIM), jnp.float32),
        compiler_params=pltpu.CompilerParams(
            dimension_semantics=("arbitrary",),
        ),
    )(sp, values, tim_emb, w2, b2)
    return out
